# baseline (speedup 1.0000x reference)
import jax
import jax.numpy as jnp
from jax import lax
from jax.experimental import pallas as pl
from jax.experimental.pallas import tpu as pltpu
from jax.experimental.pallas import tpu_sc as plsc

_N = 10000
_E = 320000
_B = 256
_H = 128
_L = 4

_CH = 128
_NC = 2
_NS = 16
_NW = _NC * _NS
_NCHUNKS = _E // _CH
_NPAD = 10240
_RPO = _NPAD // _NW
_CAP = 320128
_AGR = _RPO + 8
_SB = 40
_PSUB = 256
_PR = 1280
_PB = _E // _PR
_BN_ROWS = 1000
_NBLK = _N // _BN_ROWS


def _plan_body(d_ref, fi_ref, dl_ref, cnt_ref):
    i = pl.program_id(0)

    @pl.when(i == 0)
    def _():
        cnt_ref[...] = jnp.zeros_like(cnt_ref)

    rows_i = lax.broadcasted_iota(jnp.int32, (_PSUB, _PSUB), 0)
    cols_i = lax.broadcasted_iota(jnp.int32, (_PSUB, _PSUB), 1)
    tril = (rows_i >= cols_i).astype(jnp.bfloat16)
    base = cnt_ref[0:1, :]
    for s in range(_PR // _PSUB):
        d = d_ref[0, s * _PSUB:(s + 1) * _PSUB, :]
        owner = lax.shift_right_logical(d * 6554, 21)
        dl = d - owner * 320
        cols = lax.broadcasted_iota(jnp.int32, (_PSUB, _H), 1)
        oh = (owner == cols).astype(jnp.float32)
        ranks = jnp.dot(tril, oh.astype(jnp.bfloat16),
                        preferred_element_type=jnp.float32)
        fi_f = jnp.sum((ranks - 1.0 + base) * oh, axis=1, keepdims=True)
        fi = (owner.astype(jnp.float32) * float(_CAP) + fi_f).astype(jnp.int32)
        fi_ref[0, s * _PSUB:(s + 1) * _PSUB, :] = fi
        dl_ref[0, s * _PSUB:(s + 1) * _PSUB, :] = dl
        base = base + jnp.sum(oh, axis=0, keepdims=True)
    cnt_ref[0:1, :] = base


_tc_plan = pl.pallas_call(
    _plan_body,
    grid=(_PB,),
    in_specs=[pl.BlockSpec((1, _PR, 1), lambda i: (i, 0, 0))],
    out_specs=[pl.BlockSpec((1, _PR, 1), lambda i: (i, 0, 0)),
               pl.BlockSpec((1, _PR, 1), lambda i: (i, 0, 0)),
               pl.BlockSpec((8, _H), lambda i: (0, 0))],
    out_shape=[jax.ShapeDtypeStruct((_PB, _PR, 1), jnp.int32),
               jax.ShapeDtypeStruct((_PB, _PR, 1), jnp.int32),
               jax.ShapeDtypeStruct((8, _H), jnp.float32)],
)


def _sc_route_body(src_hbm, dl_hbm, fi_hbm, e0_hbm, e1_hbm, e2_hbm,
                   cnt_hbm, ti_hbm, tf_hbm,
                   r_src, r_dl, r_e0, r_e1, r_e2,
                   s_v, d_v, f_v, e0_v, e1_v, e2_v,
                   cnt_v, ti_v, tf_v, fp_v,
                   sem_in, sem_sc):
    cid = lax.axis_index("c")
    sid = lax.axis_index("s")
    wid = sid * _NC + cid

    pltpu.sync_copy(cnt_hbm, cnt_v)
    pltpu.sync_copy(ti_hbm, ti_v)
    pltpu.sync_copy(tf_hbm, tf_v)

    nfull = _NCHUNKS // _NW
    extra = _NCHUNKS - nfull * _NW
    niter = nfull + jnp.where(wid < extra, 1, 0)

    def start_in(c):
        b = lax.rem(c, 2)
        base = (c * _NW + wid) * _CH
        sl = pl.ds(base, _CH)
        pltpu.async_copy(src_hbm.at[sl], s_v.at[b], sem_in.at[b])
        pltpu.async_copy(dl_hbm.at[sl], d_v.at[b], sem_in.at[b])
        pltpu.async_copy(fi_hbm.at[sl], f_v.at[b], sem_in.at[b])
        pltpu.async_copy(e0_hbm.at[sl], e0_v.at[b], sem_in.at[b])
        pltpu.async_copy(e1_hbm.at[sl], e1_v.at[b], sem_in.at[b])
        pltpu.async_copy(e2_hbm.at[sl], e2_v.at[b], sem_in.at[b])

    def wait_in(b):
        sl = pl.ds(0, _CH)
        pltpu.make_async_copy(src_hbm.at[sl], s_v.at[b], sem_in.at[b]).wait()
        pltpu.make_async_copy(dl_hbm.at[sl], d_v.at[b], sem_in.at[b]).wait()
        pltpu.make_async_copy(fi_hbm.at[sl], f_v.at[b], sem_in.at[b]).wait()
        pltpu.make_async_copy(e0_hbm.at[sl], e0_v.at[b], sem_in.at[b]).wait()
        pltpu.make_async_copy(e1_hbm.at[sl], e1_v.at[b], sem_in.at[b]).wait()
        pltpu.make_async_copy(e2_hbm.at[sl], e2_v.at[b], sem_in.at[b]).wait()

    def start_sc(b):
        idx = f_v.at[b]
        pltpu.async_copy(s_v.at[b], r_src.at[idx], sem_sc.at[b])
        pltpu.async_copy(d_v.at[b], r_dl.at[idx], sem_sc.at[b])
        pltpu.async_copy(e0_v.at[b], r_e0.at[idx], sem_sc.at[b])
        pltpu.async_copy(e1_v.at[b], r_e1.at[idx], sem_sc.at[b])
        pltpu.async_copy(e2_v.at[b], r_e2.at[idx], sem_sc.at[b])

    def wait_sc(b):
        idx = f_v.at[b]
        pltpu.make_async_copy(s_v.at[b], r_src.at[idx], sem_sc.at[b]).wait()
        pltpu.make_async_copy(d_v.at[b], r_dl.at[idx], sem_sc.at[b]).wait()
        pltpu.make_async_copy(e0_v.at[b], r_e0.at[idx], sem_sc.at[b]).wait()
        pltpu.make_async_copy(e1_v.at[b], r_e1.at[idx], sem_sc.at[b]).wait()
        pltpu.make_async_copy(e2_v.at[b], r_e2.at[idx], sem_sc.at[b]).wait()

    start_in(0)

    def body(i, carry):
        b = lax.rem(i, 2)

        @pl.when(i + 1 < niter)
        def _():
            @pl.when(i >= 1)
            def _():
                wait_sc(1 - b)
            start_in(i + 1)

        wait_in(b)
        start_sc(b)
        return carry

    lax.fori_loop(0, niter, body, 0)
    wait_sc(lax.rem(niter - 1, 2))

    @pl.when(niter >= 2)
    def _():
        wait_sc(lax.rem(niter, 2))

    n_o = cnt_v[pl.ds(wid, 16)][0]
    for g in range(8):
        fp_v[g * 16:(g + 1) * 16] = (
            (n_o + wid * _CAP + g * 16)
            + lax.broadcasted_iota(jnp.int32, (16,), 0))
    pltpu.async_copy(ti_v.at[0], r_src.at[fp_v], sem_sc.at[0])
    pltpu.async_copy(ti_v.at[1], r_dl.at[fp_v], sem_sc.at[0])
    pltpu.async_copy(tf_v.at[0], r_e0.at[fp_v], sem_sc.at[0])
    pltpu.async_copy(tf_v.at[0], r_e1.at[fp_v], sem_sc.at[0])
    pltpu.async_copy(tf_v.at[0], r_e2.at[fp_v], sem_sc.at[0])
    pltpu.make_async_copy(ti_v.at[0], r_src.at[fp_v], sem_sc.at[0]).wait()
    pltpu.make_async_copy(ti_v.at[1], r_dl.at[fp_v], sem_sc.at[0]).wait()
    pltpu.make_async_copy(tf_v.at[0], r_e0.at[fp_v], sem_sc.at[0]).wait()
    pltpu.make_async_copy(tf_v.at[0], r_e1.at[fp_v], sem_sc.at[0]).wait()
    pltpu.make_async_copy(tf_v.at[0], r_e2.at[fp_v], sem_sc.at[0]).wait()


_RTOT = _NW * _CAP + _SB * _CH

_sc_route = pl.kernel(
    _sc_route_body,
    out_type=[jax.ShapeDtypeStruct((_RTOT,), jnp.int32),
              jax.ShapeDtypeStruct((_RTOT,), jnp.int32),
              jax.ShapeDtypeStruct((_RTOT,), jnp.float32),
              jax.ShapeDtypeStruct((_RTOT,), jnp.float32),
              jax.ShapeDtypeStruct((_RTOT,), jnp.float32)],
    mesh=plsc.VectorSubcoreMesh(core_axis_name="c", subcore_axis_name="s"),
    scratch_types=[
        pltpu.VMEM((2, _CH), jnp.int32),
        pltpu.VMEM((2, _CH), jnp.int32),
        pltpu.VMEM((2, _CH), jnp.int32),
        pltpu.VMEM((2, _CH), jnp.float32),
        pltpu.VMEM((2, _CH), jnp.float32),
        pltpu.VMEM((2, _CH), jnp.float32),
        pltpu.VMEM((_CH,), jnp.int32),
        pltpu.VMEM((2, _CH), jnp.int32),
        pltpu.VMEM((1, _CH), jnp.float32),
        pltpu.VMEM((_CH,), jnp.int32),
        pltpu.SemaphoreType.DMA((2,)),
        pltpu.SemaphoreType.DMA((2,)),
    ],
    name="sc_gine_route",
)


def _sc_agg_body(h_hbm, rs_hbm, rd_hbm, re0_hbm, re1_hbm, re2_hbm,
                 cnt_hbm, wb_hbm, out_hbm,
                 s_a, d_a, e0_a, e1_a, e2_a, cnt_v, wb_v, rows_v, agg_v,
                 sem_g):
    cid = lax.axis_index("c")
    sid = lax.axis_index("s")
    wid = sid * _NC + cid

    pltpu.sync_copy(cnt_hbm, cnt_v)
    pltpu.sync_copy(wb_hbm, wb_v)

    def zrow(r, carry):
        for j in range(_H // 16):
            agg_v[r, pl.ds(j * 16, 16)] = jnp.zeros((16,), jnp.float32)
        return carry

    lax.fori_loop(0, _AGR, zrow, 0)

    w_sl = [[wb_v[i, 16 * j:16 * (j + 1)] for j in range(_H // 16)]
            for i in range(4)]

    n_o = cnt_v[pl.ds(wid, 16)][0]
    nch = lax.div(n_o + (_CH - 1), _CH)
    nr = lax.div(nch + (_SB - 1), _SB)
    obase = wid * _CAP

    def start_gather(k):
        b = lax.rem(k, 2)
        pltpu.async_copy(h_hbm.at[s_a.at[pl.ds(k * _CH, _CH)]],
                         rows_v.at[b], sem_g.at[b])

    def wait_gather(k):
        b = lax.rem(k, 2)
        pltpu.make_async_copy(h_hbm.at[s_a.at[pl.ds(0, _CH)]],
                              rows_v.at[b], sem_g.at[b]).wait()

    def compute(k):
        b = lax.rem(k, 2)

        def group_body(g, c2):
            gb = k * _CH + g * 16
            a0v = e0_a[pl.ds(gb, 16)]
            a1v = e1_a[pl.ds(gb, 16)]
            a2v = e2_a[pl.ds(gb, 16)]
            dlv = d_a[pl.ds(gb, 16)]
            for kk in range(16):
                a0 = a0v[kk]
                a1 = a1v[kk]
                a2 = a2v[kk]
                dl = dlv[kk]
                e = g * 16 + kk
                for j in range(_H // 16):
                    sl = pl.ds(j * 16, 16)
                    v = rows_v[b, e, sl] + (a0 * w_sl[0][j] + a1 * w_sl[1][j]
                                            + a2 * w_sl[2][j] + w_sl[3][j])
                    agg_v[dl, sl] += jnp.maximum(v, 0.0)
            return c2

        lax.fori_loop(0, _CH // 16, group_body, 0)

    def round_body(r, carry):
        rbase = obase + r * (_SB * _CH)
        sl = pl.ds(rbase, _SB * _CH)
        pltpu.sync_copy(rs_hbm.at[sl], s_a)
        pltpu.sync_copy(rd_hbm.at[sl], d_a)
        pltpu.sync_copy(re0_hbm.at[sl], e0_a)
        pltpu.sync_copy(re1_hbm.at[sl], e1_a)
        pltpu.sync_copy(re2_hbm.at[sl], e2_a)
        kn = jnp.minimum(_SB, nch - r * _SB)
        start_gather(0)

        def kbody(k, c2):
            @pl.when(k + 1 < kn)
            def _():
                start_gather(k + 1)

            wait_gather(k)
            compute(k)
            return c2

        lax.fori_loop(0, kn, kbody, 0)
        return carry

    lax.fori_loop(0, nr, round_body, 0)
    pltpu.sync_copy(agg_v.at[pl.ds(0, _RPO), :],
                    out_hbm.at[pl.ds(wid * _RPO, _RPO), :])


_sc_agg = pl.kernel(
    _sc_agg_body,
    out_type=jax.ShapeDtypeStruct((_NPAD, _H), jnp.float32),
    mesh=plsc.VectorSubcoreMesh(core_axis_name="c", subcore_axis_name="s"),
    scratch_types=[
        pltpu.VMEM((_SB * _CH,), jnp.int32),
        pltpu.VMEM((_SB * _CH,), jnp.int32),
        pltpu.VMEM((_SB * _CH,), jnp.float32),
        pltpu.VMEM((_SB * _CH,), jnp.float32),
        pltpu.VMEM((_SB * _CH,), jnp.float32),
        pltpu.VMEM((_CH,), jnp.int32),
        pltpu.VMEM((4, _H), jnp.float32),
        pltpu.VMEM((2, _CH, _H), jnp.float32),
        pltpu.VMEM((_AGR, _H), jnp.float32),
        pltpu.SemaphoreType.DMA((2,)),
    ],
    name="sc_gine_agg",
)


def _lin_body(x_ref, w_ref, b_ref, o_ref):
    o_ref[...] = (jnp.dot(x_ref[...], w_ref[...],
                          preferred_element_type=jnp.float32) + b_ref[...])


def _mlp_body(h_ref, a_ref, w1_ref, b1_ref, w2_ref, b2_ref, t_ref, s_ref):
    i = pl.program_id(0)
    z = h_ref[...] + a_ref[...]
    u = jnp.maximum(jnp.dot(z, w1_ref[...],
                            preferred_element_type=jnp.float32) + b1_ref[...],
                    0.0)
    t = jnp.dot(u, w2_ref[...], preferred_element_type=jnp.float32) + b2_ref[...]
    t_ref[...] = t

    @pl.when(i == 0)
    def _():
        s_ref[...] = jnp.zeros_like(s_ref)

    s_ref[0:1, :] += jnp.sum(t, axis=0, keepdims=True)
    s_ref[1:2, :] += jnp.sum(t * t, axis=0, keepdims=True)


def _bn_body(t_ref, s_ref, g_ref, b_ref, o_ref):
    mean = s_ref[0:1, :] * (1.0 / _N)
    var = s_ref[1:2, :] * (1.0 / _N) - mean * mean
    inv = lax.rsqrt(var + 1e-5)
    o_ref[...] = jnp.maximum(
        g_ref[...] * (t_ref[...] - mean) * inv + b_ref[...], 0.0)


def _pool_body(h_ref, b_ref, o_ref, sums, counts):
    i = pl.program_id(0)

    @pl.when(i == 0)
    def _():
        sums[...] = jnp.zeros_like(sums)
        counts[...] = jnp.zeros_like(counts)

    bvals = b_ref[0]
    ids = lax.broadcasted_iota(jnp.int32, (_B, _BN_ROWS), 0)
    onehot = (bvals == ids).astype(jnp.float32)
    sums[...] += jnp.dot(onehot, h_ref[...], preferred_element_type=jnp.float32)
    counts[...] += jnp.sum(onehot, axis=1, keepdims=True)

    @pl.when(i == pl.num_programs(0) - 1)
    def _():
        o_ref[...] = sums[...] / jnp.maximum(counts[...], 1.0)


_row_spec = pl.BlockSpec((_BN_ROWS, _H), lambda i: (i, 0))
_full_mat = pl.BlockSpec((_H, _H), lambda i: (0, 0))
_full_vec = pl.BlockSpec((1, _H), lambda i: (0, 0))
_stat_spec = pl.BlockSpec((2, _H), lambda i: (0, 0))

_tc_linear = pl.pallas_call(
    _lin_body,
    grid=(_NBLK,),
    in_specs=[_row_spec, _full_mat, _full_vec],
    out_specs=_row_spec,
    out_shape=jax.ShapeDtypeStruct((_N, _H), jnp.float32),
)

_tc_mlp = pl.pallas_call(
    _mlp_body,
    grid=(_NBLK,),
    in_specs=[_row_spec, _row_spec,
              _full_mat, _full_vec, _full_mat, _full_vec],
    out_specs=[_row_spec, _stat_spec],
    out_shape=[jax.ShapeDtypeStruct((_N, _H), jnp.float32),
               jax.ShapeDtypeStruct((2, _H), jnp.float32)],
)

_tc_bn = pl.pallas_call(
    _bn_body,
    grid=(_NBLK,),
    in_specs=[_row_spec, _stat_spec, _full_vec, _full_vec],
    out_specs=_row_spec,
    out_shape=jax.ShapeDtypeStruct((_N, _H), jnp.float32),
)

_tc_pool = pl.pallas_call(
    _pool_body,
    grid=(_NBLK,),
    in_specs=[_row_spec,
              pl.BlockSpec((1, 1, _BN_ROWS), lambda i: (i, 0, 0))],
    out_specs=pl.BlockSpec((_B, _H), lambda i: (0, 0)),
    out_shape=jax.ShapeDtypeStruct((_B, _H), jnp.float32),
    scratch_shapes=[pltpu.VMEM((_B, _H), jnp.float32),
                    pltpu.VMEM((_B, 1), jnp.float32)],
)


def kernel(x, edge_index, edge_attr, batch, W_in, b_in, W_edge, b_edge,
           W1, b1, W2, b2, gamma, beta):
    src = edge_index[0]
    dst = edge_index[1]
    ea_t = edge_attr.T

    fi, dl, cnt = _tc_plan(dst.reshape(_PB, _PR, 1))
    cnt_i = jnp.where(lax.broadcasted_iota(jnp.int32, (_CH,), 0) < _NW,
                      cnt[0].astype(jnp.int32), 0)
    ti = jnp.stack([jnp.zeros((_CH,), jnp.int32),
                    jnp.full((_CH,), _RPO, jnp.int32)])
    tf = jnp.zeros((1, _CH), jnp.float32)
    r_src, r_dl, r_e0, r_e1, r_e2 = _sc_route(
        src, dl.reshape(_E), fi.reshape(_E), ea_t[0], ea_t[1], ea_t[2],
        cnt_i, ti, tf)

    nd = x.shape[1]
    x_p = jnp.zeros((_N, _H), jnp.float32).at[:, :nd].set(x)
    w_p = jnp.zeros((_H, _H), jnp.float32).at[:nd, :].set(W_in)
    h = _tc_linear(x_p, w_p, b_in.reshape(1, _H))

    for l in range(_L):
        wb = jnp.concatenate([W_edge[l], b_edge[l].reshape(1, _H)], axis=0)
        agg = _sc_agg(h, r_src, r_dl, r_e0, r_e1, r_e2, cnt_i, wb)
        t, stats = _tc_mlp(h, agg[:_N],
                           W1[l], b1[l].reshape(1, _H),
                           W2[l], b2[l].reshape(1, _H))
        h = _tc_bn(t, stats, gamma[l].reshape(1, _H), beta[l].reshape(1, _H))

    return _tc_pool(h, batch.reshape(_NBLK, 1, _BN_ROWS))


# speedup vs baseline: 2.3714x; 2.3714x over previous
import jax
import jax.numpy as jnp
from jax import lax
from jax.experimental import pallas as pl
from jax.experimental.pallas import tpu as pltpu
from jax.experimental.pallas import tpu_sc as plsc

_N = 10000
_E = 320000
_B = 256
_H = 128
_L = 4

_CH = 128
_NC = 2
_NS = 16
_NW = _NC * _NS
_NCHUNKS = _E // _CH
_NPAD = 10240
_RPT = _NPAD // _NS
_BN_ROWS = 1000
_NBLK = _N // _BN_ROWS


def _sc_message_body(h_hbm, src_hbm, dst_hbm, ea0_hbm, ea1_hbm, ea2_hbm,
                     wb_hbm, z_hbm, out_hbm,
                     src_v, dst_v, ea0_v, ea1_v, ea2_v, wb_v, rows_v, agg_sh,
                     sem):
    cid = lax.axis_index("c")
    sid = lax.axis_index("s")
    wid = sid * _NC + cid

    pltpu.sync_copy(z_hbm, agg_sh.at[pl.ds(sid * _RPT, _RPT), :])
    pltpu.sync_copy(wb_hbm, wb_v)
    plsc.subcore_barrier()

    nfull = _NCHUNKS // _NW
    extra = _NCHUNKS - nfull * _NW
    niter = nfull + jnp.where(wid < extra, 1, 0)

    w_sl = [[wb_v[i, 16 * j:16 * (j + 1)] for j in range(_H // 16)]
            for i in range(4)]

    def chunk_body(i, carry):
        base = (i * _NW + wid) * _CH
        pltpu.sync_copy(src_hbm.at[pl.ds(base, _CH)], src_v)
        pltpu.sync_copy(dst_hbm.at[pl.ds(base, _CH)], dst_v)
        pltpu.sync_copy(ea0_hbm.at[pl.ds(base, _CH)], ea0_v)
        pltpu.sync_copy(ea1_hbm.at[pl.ds(base, _CH)], ea1_v)
        pltpu.sync_copy(ea2_hbm.at[pl.ds(base, _CH)], ea2_v)
        pltpu.async_copy(h_hbm.at[src_v], rows_v, sem).wait()

        def group_body(g, c2):
            gb = g * 16
            a0v = ea0_v[pl.ds(gb, 16)]
            a1v = ea1_v[pl.ds(gb, 16)]
            a2v = ea2_v[pl.ds(gb, 16)]
            for k in range(16):
                a0 = a0v[k]
                a1 = a1v[k]
                a2 = a2v[k]
                e = gb + k
                for j in range(_H // 16):
                    sl = pl.ds(j * 16, 16)
                    v = rows_v[e, sl] + (a0 * w_sl[0][j] + a1 * w_sl[1][j]
                                         + a2 * w_sl[2][j] + w_sl[3][j])
                    rows_v[e, sl] = jnp.maximum(v, 0.0)
            return c2

        lax.fori_loop(0, _CH // 16, group_body, 0)
        pltpu.sync_copy(rows_v, agg_sh.at[dst_v], add=True)
        return carry

    lax.fori_loop(0, niter, chunk_body, 0)
    plsc.subcore_barrier()
    pltpu.sync_copy(agg_sh.at[pl.ds(sid * _RPT, _RPT), :],
                    out_hbm.at[cid, pl.ds(sid * _RPT, _RPT), :])


_sc_message = pl.kernel(
    _sc_message_body,
    out_type=jax.ShapeDtypeStruct((_NC, _NPAD, _H), jnp.float32),
    mesh=plsc.VectorSubcoreMesh(core_axis_name="c", subcore_axis_name="s"),
    scratch_types=[
        pltpu.VMEM((_CH,), jnp.int32),
        pltpu.VMEM((_CH,), jnp.int32),
        pltpu.VMEM((_CH,), jnp.float32),
        pltpu.VMEM((_CH,), jnp.float32),
        pltpu.VMEM((_CH,), jnp.float32),
        pltpu.VMEM((4, _H), jnp.float32),
        pltpu.VMEM((_CH, _H), jnp.float32),
        pltpu.VMEM_SHARED((_NPAD, _H), jnp.float32),
        pltpu.SemaphoreType.DMA,
    ],
    name="sc_gine_message",
)


def _lin_body(x_ref, w_ref, b_ref, o_ref):
    o_ref[...] = (jnp.dot(x_ref[...], w_ref[...],
                          preferred_element_type=jnp.float32) + b_ref[...])


def _mlp_body(h_ref, a0_ref, a1_ref, w1_ref, b1_ref, w2_ref, b2_ref,
              t_ref, s_ref):
    i = pl.program_id(0)
    z = h_ref[...] + a0_ref[...] + a1_ref[...]
    u = jnp.maximum(jnp.dot(z, w1_ref[...],
                            preferred_element_type=jnp.float32) + b1_ref[...],
                    0.0)
    t = jnp.dot(u, w2_ref[...], preferred_element_type=jnp.float32) + b2_ref[...]
    t_ref[...] = t

    @pl.when(i == 0)
    def _():
        s_ref[...] = jnp.zeros_like(s_ref)

    s_ref[0:1, :] += jnp.sum(t, axis=0, keepdims=True)
    s_ref[1:2, :] += jnp.sum(t * t, axis=0, keepdims=True)


def _bn_body(t_ref, s_ref, g_ref, b_ref, o_ref):
    mean = s_ref[0:1, :] * (1.0 / _N)
    var = s_ref[1:2, :] * (1.0 / _N) - mean * mean
    inv = lax.rsqrt(var + 1e-5)
    o_ref[...] = jnp.maximum(
        g_ref[...] * (t_ref[...] - mean) * inv + b_ref[...], 0.0)


def _pool_body(h_ref, b_ref, o_ref, sums, counts):
    i = pl.program_id(0)

    @pl.when(i == 0)
    def _():
        sums[...] = jnp.zeros_like(sums)
        counts[...] = jnp.zeros_like(counts)

    bvals = b_ref[0]
    ids = lax.broadcasted_iota(jnp.int32, (_B, _BN_ROWS), 0)
    onehot = (bvals == ids).astype(jnp.float32)
    sums[...] += jnp.dot(onehot, h_ref[...], preferred_element_type=jnp.float32)
    counts[...] += jnp.sum(onehot, axis=1, keepdims=True)

    @pl.when(i == pl.num_programs(0) - 1)
    def _():
        o_ref[...] = sums[...] / jnp.maximum(counts[...], 1.0)


_row_spec = pl.BlockSpec((_BN_ROWS, _H), lambda i: (i, 0))
_full_mat = pl.BlockSpec((_H, _H), lambda i: (0, 0))
_full_vec = pl.BlockSpec((1, _H), lambda i: (0, 0))
_stat_spec = pl.BlockSpec((2, _H), lambda i: (0, 0))

_tc_linear = pl.pallas_call(
    _lin_body,
    grid=(_NBLK,),
    in_specs=[_row_spec, _full_mat, _full_vec],
    out_specs=_row_spec,
    out_shape=jax.ShapeDtypeStruct((_N, _H), jnp.float32),
)

_tc_mlp = pl.pallas_call(
    _mlp_body,
    grid=(_NBLK,),
    in_specs=[_row_spec, _row_spec, _row_spec,
              _full_mat, _full_vec, _full_mat, _full_vec],
    out_specs=[_row_spec, _stat_spec],
    out_shape=[jax.ShapeDtypeStruct((_N, _H), jnp.float32),
               jax.ShapeDtypeStruct((2, _H), jnp.float32)],
)

_tc_bn = pl.pallas_call(
    _bn_body,
    grid=(_NBLK,),
    in_specs=[_row_spec, _stat_spec, _full_vec, _full_vec],
    out_specs=_row_spec,
    out_shape=jax.ShapeDtypeStruct((_N, _H), jnp.float32),
)

_tc_pool = pl.pallas_call(
    _pool_body,
    grid=(_NBLK,),
    in_specs=[_row_spec,
              pl.BlockSpec((1, 1, _BN_ROWS), lambda i: (i, 0, 0))],
    out_specs=pl.BlockSpec((_B, _H), lambda i: (0, 0)),
    out_shape=jax.ShapeDtypeStruct((_B, _H), jnp.float32),
    scratch_shapes=[pltpu.VMEM((_B, _H), jnp.float32),
                    pltpu.VMEM((_B, 1), jnp.float32)],
)


def kernel(x, edge_index, edge_attr, batch, W_in, b_in, W_edge, b_edge,
           W1, b1, W2, b2, gamma, beta):
    src = edge_index[0]
    dst = edge_index[1]
    ea_t = edge_attr.T
    zrows = jnp.zeros((_RPT, _H), jnp.float32)

    nd = x.shape[1]
    x_p = jnp.zeros((_N, _H), jnp.float32).at[:, :nd].set(x)
    w_p = jnp.zeros((_H, _H), jnp.float32).at[:nd, :].set(W_in)
    h = _tc_linear(x_p, w_p, b_in.reshape(1, _H))

    for l in range(_L):
        wb = jnp.concatenate([W_edge[l], b_edge[l].reshape(1, _H)], axis=0)
        agg = _sc_message(h, src, dst, ea_t[0], ea_t[1], ea_t[2], wb, zrows)
        t, stats = _tc_mlp(h, agg[0, :_N], agg[1, :_N],
                           W1[l], b1[l].reshape(1, _H),
                           W2[l], b2[l].reshape(1, _H))
        h = _tc_bn(t, stats, gamma[l].reshape(1, _H), beta[l].reshape(1, _H))

    return _tc_pool(h, batch.reshape(_NBLK, 1, _BN_ROWS))
